# baseline (speedup 1.0000x reference)
import functools

import jax
import jax.numpy as jnp
from jax.experimental import pallas as pl
from jax.experimental.pallas import tpu as pltpu

_EPAD = 128


def _body(E, S, DB, HB, sp_ref, xc_ref, w1_ref, b1_ref, w2_ref, b2_ref,
          out_ref, xbf_ref, log_ref):
    s = pl.program_id(0)
    ns = pl.num_programs(0)

    @pl.when(s < S)
    def _():
        xbf_ref[s] = xc_ref[...].astype(jnp.bfloat16)

    @pl.when(s >= S)
    def _():
        j = s - S
        w1b = w1_ref[...].astype(jnp.bfloat16)
        pre = jnp.zeros((xbf_ref.shape[1], HB), jnp.float32)
        for k in range(S):
            pre = pre + jnp.dot(xbf_ref[k],
                                w1b[k * DB:(k + 1) * DB, :],
                                preferred_element_type=jnp.float32)
        h = jax.nn.gelu(pre + b1_ref[...])
        w2b = w2_ref[...].astype(jnp.bfloat16)
        w2pb = jnp.concatenate(
            [w2b, jnp.zeros((HB, _EPAD - w2b.shape[1]), jnp.bfloat16)], axis=1)
        plog = jnp.dot(h.astype(jnp.bfloat16), w2pb,
                       preferred_element_type=jnp.float32)

        @pl.when(j == 0)
        def _():
            b2p = jnp.concatenate(
                [b2_ref[...],
                 jnp.full((1, _EPAD - b2_ref.shape[1]), -1e30, jnp.float32)],
                axis=1)
            log_ref[...] = plog + b2p

        @pl.when(j != 0)
        def _():
            log_ref[...] = log_ref[...] + plog

        @pl.when(s == ns - 1)
        def _():
            lg = log_ref[...]
            m = jnp.max(lg)
            rows = jax.lax.broadcasted_iota(jnp.int32, lg.shape, 0)
            cols = jax.lax.broadcasted_iota(jnp.int32, lg.shape, 1)
            flat = rows * E + cols
            idx = jnp.min(jnp.where(lg == m, flat, jnp.int32(2**30)))
            out_ref[...] = jnp.broadcast_to(idx, out_ref.shape)


def _gather_body(sp0_ref, sp1_ref, tab_ref, out_ref):
    r = sp1_ref[0] % 8
    out_ref[...] = tab_ref[0, pl.ds(r, 1), :]


def kernel(predicate, W1, b1, W2, b2, expert_tables, input):
    T, D = predicate.shape
    H = W1.shape[1]
    E = W2.shape[1]
    n_tab, ROWS, ED = expert_tables.shape

    DB = 512
    S = D // DB
    HB = 512
    J = H // HB

    b1r = b1.reshape(1, H)
    b2r = b2.reshape(1, E)
    sp = jnp.asarray(input, jnp.int32).reshape(1)

    grid_spec = pltpu.PrefetchScalarGridSpec(
        num_scalar_prefetch=1,
        grid=(S + J,),
        in_specs=[
            pl.BlockSpec((T, DB), lambda s, sp: (0, jnp.minimum(s, S - 1))),
            pl.BlockSpec((D, HB),
                         lambda s, sp: (0, jnp.clip(s - S, 0, J - 1))),
            pl.BlockSpec((1, HB),
                         lambda s, sp: (0, jnp.clip(s - S, 0, J - 1))),
            pl.BlockSpec((HB, E),
                         lambda s, sp: (jnp.clip(s - S, 0, J - 1), 0)),
            pl.BlockSpec((1, E), lambda s, sp: (0, 0)),
        ],
        out_specs=pl.BlockSpec((1, 128), lambda s, sp: (0, 0)),
        scratch_shapes=[
            pltpu.VMEM((S, T, DB), jnp.bfloat16),
            pltpu.VMEM((T, _EPAD), jnp.float32),
        ],
    )

    idx = pl.pallas_call(
        functools.partial(_body, E, S, DB, HB),
        grid_spec=grid_spec,
        out_shape=jax.ShapeDtypeStruct((1, 128), jnp.int32),
        compiler_params=pltpu.CompilerParams(
            dimension_semantics=("arbitrary",),
        ),
    )(sp, predicate, W1, b1r, W2, b2r)

    gather_spec = pltpu.PrefetchScalarGridSpec(
        num_scalar_prefetch=2,
        grid=(1,),
        in_specs=[
            pl.BlockSpec((1, 8, ED),
                         lambda i, sp0, sp1: (sp0[0], sp1[0] // 8, 0)),
        ],
        out_specs=pl.BlockSpec((1, ED), lambda i, sp0, sp1: (0, 0)),
    )
    out = pl.pallas_call(
        _gather_body,
        grid_spec=gather_spec,
        out_shape=jax.ShapeDtypeStruct((1, ED), jnp.float32),
    )(sp, idx.reshape(128), expert_tables)
    return out.reshape(ED)
